# baseline (speedup 1.0000x reference)
import functools

import jax
import jax.numpy as jnp
from jax import lax
from jax.experimental import pallas as pl
from jax.experimental.pallas import tpu as pltpu
from jax.experimental.pallas import tpu_sc as plsc

_B = 1024
_V = 100000
_L = 16
_NC = 2
_NS = 16
_PER_TILE = _B // (_NC * _NS)
_CHUNKS = _PER_TILE // _L


def _loss_body(inT_hbm, tgt_hbm, out_hbm, tgt_v, val_v, all_v, red_v, shared, sem):
    cid = lax.axis_index("c")
    sid = lax.axis_index("s")
    base = cid * (_NS * _PER_TILE) + sid * _PER_TILE

    pltpu.sync_copy(tgt_hbm.at[pl.ds(base, _PER_TILE)], tgt_v)

    col0 = pl.multiple_of(cid * (_NS * _PER_TILE) + (sid // 4) * 128, 128)
    copies = []
    for j in range(_CHUNKS):
        rg0v = lax.shift_left(
            lax.shift_right_logical(tgt_v[pl.ds(j * _L, _L)], 3), 3
        )
        for i in range(_L):
            k = j * _L + i
            rg0 = pl.multiple_of(rg0v[i], 8)
            copies.append(
                pltpu.make_async_copy(
                    inT_hbm.at[pl.ds(rg0, 8), pl.ds(col0, 128)], val_v.at[k], sem
                )
            )
    for c in copies:
        c.start()
    for c in copies:
        c.wait()

    acc = jnp.zeros((_L,), jnp.float32)
    cbase = (sid % 4) * _PER_TILE
    for j in range(_CHUNKS):
        blk = lax.iota(jnp.int32, _L) + (j * _L)
        row = lax.bitwise_and(tgt_v[pl.ds(j * _L, _L)], 7)
        col = lax.iota(jnp.int32, _L) + (cbase + j * _L)
        acc = acc + plsc.load_gather(val_v, [blk, row, col])
    red_v[...] = acc
    pltpu.sync_copy(red_v, shared.at[pl.ds(sid * _L, _L)])
    plsc.subcore_barrier()

    @pl.when(sid == 0)
    def _():
        pltpu.sync_copy(shared, all_v)
        tot = all_v[pl.ds(0, _L)]
        for i in range(1, _NS):
            tot = tot + all_v[pl.ds(i * _L, _L)]
        red_v[...] = jnp.full((_L,), jnp.sum(tot), jnp.float32)
        pltpu.sync_copy(red_v, out_hbm.at[cid])


@jax.jit
def _loss(inT, tgt):
    mesh = plsc.VectorSubcoreMesh(
        core_axis_name="c", subcore_axis_name="s", num_cores=_NC
    )
    return pl.kernel(
        _loss_body,
        out_type=jax.ShapeDtypeStruct((_NC, _L), jnp.float32),
        mesh=mesh,
        scratch_types=[
            pltpu.VMEM((_PER_TILE,), jnp.int32),
            pltpu.VMEM((_PER_TILE, 8, 128), jnp.float32),
            pltpu.VMEM((_NS * _L,), jnp.float32),
            pltpu.VMEM((_L,), jnp.float32),
            pltpu.VMEM_SHARED((_NS * _L,), jnp.float32),
            pltpu.SemaphoreType.DMA,
        ],
        compiler_params=pltpu.CompilerParams(needs_layout_passes=False),
    )(inT, tgt)


def kernel(input_, target):
    out = _loss(input_.T, target.astype(jnp.int32))
    return 1.0 - (out[0, 0] + out[1, 0]) * (1.0 / _B)


# speedup vs baseline: 1.1762x; 1.1762x over previous
import functools

import jax
import jax.numpy as jnp
from jax import lax
from jax.experimental import pallas as pl
from jax.experimental.pallas import tpu as pltpu
from jax.experimental.pallas import tpu_sc as plsc

_B = 1024
_V = 100000
_L = 16
_NS = 16
_PER_TILE = _B // _NS
_CHUNKS = _PER_TILE // _L


def _loss_body(inT_hbm, tgt_hbm, out_hbm, tgt_v, val_v, all_v, red_v, shared, sem):
    sid = lax.axis_index("s")
    base = sid * _PER_TILE

    pltpu.sync_copy(tgt_hbm.at[pl.ds(base, _PER_TILE)], tgt_v)

    col0 = pl.multiple_of((sid // 2) * 128, 128)

    def _fire(j, carry):
        rg0v = lax.shift_left(
            lax.shift_right_logical(tgt_v[pl.ds(j * _L, _L)], 3), 3
        )
        for i in range(_L):
            rg0 = pl.multiple_of(rg0v[i], 8)
            dst = val_v.at[pl.ds((j * _L + i) * 8, 8), :]
            pltpu.make_async_copy(
                inT_hbm.at[pl.ds(rg0, 8), pl.ds(col0, 128)], dst, sem
            ).start()
        return carry

    lax.fori_loop(0, _CHUNKS, _fire, 0)
    pltpu.make_async_copy(
        inT_hbm.at[pl.ds(0, _PER_TILE * 8), pl.ds(0, 128)], val_v, sem
    ).wait()

    acc = jnp.zeros((_L,), jnp.float32)
    cbase = (sid % 2) * _PER_TILE
    for j in range(_CHUNKS):
        blk8 = lax.shift_left(lax.iota(jnp.int32, _L) + (j * _L), 3)
        row = blk8 + lax.bitwise_and(tgt_v[pl.ds(j * _L, _L)], 7)
        col = lax.iota(jnp.int32, _L) + (cbase + j * _L)
        acc = acc + plsc.load_gather(val_v, [row, col])
    red_v[...] = acc
    pltpu.sync_copy(red_v, shared.at[pl.ds(sid * _L, _L)])
    plsc.subcore_barrier()

    @pl.when(sid == 0)
    def _():
        pltpu.sync_copy(shared, all_v)
        tot = all_v[pl.ds(0, _L)]
        for i in range(1, _NS):
            tot = tot + all_v[pl.ds(i * _L, _L)]
        res = 1.0 - jnp.sum(tot) * (1.0 / _B)
        red_v[...] = jnp.full((_L,), res, jnp.float32)
        pltpu.sync_copy(red_v, out_hbm)


@jax.jit
def _loss(inT, tgt):
    mesh = plsc.VectorSubcoreMesh(
        core_axis_name="c", subcore_axis_name="s", num_cores=1
    )
    return pl.kernel(
        _loss_body,
        out_type=jax.ShapeDtypeStruct((_L,), jnp.float32),
        mesh=mesh,
        scratch_types=[
            pltpu.VMEM((_PER_TILE,), jnp.int32),
            pltpu.VMEM((_PER_TILE * 8, 128), jnp.float32),
            pltpu.VMEM((_NS * _L,), jnp.float32),
            pltpu.VMEM((_L,), jnp.float32),
            pltpu.VMEM_SHARED((_NS * _L,), jnp.float32),
            pltpu.SemaphoreType.DMA,
        ],
        compiler_params=pltpu.CompilerParams(needs_layout_passes=False),
    )(inT, tgt)


def kernel(input_, target):
    out = _loss(input_.T, target.astype(jnp.int32))
    return out[0]


# speedup vs baseline: 1.3085x; 1.1125x over previous
import functools

import jax
import jax.numpy as jnp
from jax import lax
from jax.experimental import pallas as pl
from jax.experimental.pallas import tpu as pltpu
from jax.experimental.pallas import tpu_sc as plsc

_B = 1024
_V = 100000
_L = 16
_NS = 16
_PER_TILE = _B // _NS
_CHUNKS = _PER_TILE // _L


def _loss_body(inT_hbm, tgt_hbm, out_hbm, tgt_v, val_v, all_v, red_v, shared, sem):
    sid = lax.axis_index("s")
    base = sid * _PER_TILE

    pltpu.sync_copy(tgt_hbm.at[pl.ds(base, _PER_TILE)], tgt_v)

    col0 = pl.multiple_of((sid // 2) * 128, 128)
    pltpu.async_copy(
        inT_hbm.at[tgt_v, pl.ds(col0, 128)], val_v, sem
    ).wait()

    acc = jnp.zeros((_L,), jnp.float32)
    cbase = (sid % 2) * _PER_TILE
    for j in range(_CHUNKS):
        row = lax.iota(jnp.int32, _L) + (j * _L)
        col = lax.iota(jnp.int32, _L) + (cbase + j * _L)
        acc = acc + plsc.load_gather(val_v, [row, col])
    red_v[...] = acc
    pltpu.sync_copy(red_v, shared.at[pl.ds(sid * _L, _L)])
    plsc.subcore_barrier()

    @pl.when(sid == 0)
    def _():
        pltpu.sync_copy(shared, all_v)
        tot = all_v[pl.ds(0, _L)]
        for i in range(1, _NS):
            tot = tot + all_v[pl.ds(i * _L, _L)]
        res = 1.0 - jnp.sum(tot) * (1.0 / _B)
        red_v[...] = jnp.full((_L,), res, jnp.float32)
        pltpu.sync_copy(red_v, out_hbm)


@jax.jit
def _loss(inT, tgt):
    mesh = plsc.VectorSubcoreMesh(
        core_axis_name="c", subcore_axis_name="s", num_cores=1
    )
    return pl.kernel(
        _loss_body,
        out_type=jax.ShapeDtypeStruct((_L,), jnp.float32),
        mesh=mesh,
        scratch_types=[
            pltpu.VMEM((_PER_TILE,), jnp.int32),
            pltpu.VMEM((_PER_TILE, 128), jnp.float32),
            pltpu.VMEM((_NS * _L,), jnp.float32),
            pltpu.VMEM((_L,), jnp.float32),
            pltpu.VMEM_SHARED((_NS * _L,), jnp.float32),
            pltpu.SemaphoreType.DMA,
        ],
        compiler_params=pltpu.CompilerParams(needs_layout_passes=False),
    )(inT, tgt)


def kernel(input_, target):
    out = _loss(input_.T, target.astype(jnp.int32))
    return out[0]


# speedup vs baseline: 1.3116x; 1.0023x over previous
import functools

import jax
import jax.numpy as jnp
from jax import lax
from jax.experimental import pallas as pl
from jax.experimental.pallas import tpu as pltpu
from jax.experimental.pallas import tpu_sc as plsc

_B = 1024
_V = 100000
_L = 16
_NS = 16
_PER_TILE = _B // _NS
_CHUNKS = _PER_TILE // _L


def _loss_body(inT_hbm, tgt_hbm, out_hbm, tgt_v, val_v, all_v, red_v, shared, sem):
    sid = lax.axis_index("s")
    base = sid * _PER_TILE

    pltpu.sync_copy(tgt_hbm.at[pl.ds(base, _PER_TILE)], tgt_v)

    col0 = pl.multiple_of((sid // 2) * 128, 128)
    pltpu.async_copy(
        inT_hbm.at[tgt_v, pl.ds(col0, 128)], val_v, sem
    ).wait()

    acc = jnp.zeros((_L,), jnp.float32)
    cbase = (sid % 2) * _PER_TILE
    for j in range(_CHUNKS):
        row = lax.iota(jnp.int32, _L) + (j * _L)
        col = lax.iota(jnp.int32, _L) + (cbase + j * _L)
        acc = acc + plsc.load_gather(val_v, [row, col])
    red_v[...] = acc
    pltpu.sync_copy(red_v, shared.at[pl.ds(sid * _L, _L)])
    plsc.subcore_barrier()

    @pl.when(sid == 0)
    def _():
        pltpu.sync_copy(shared, all_v)

        def _fold(i, tot):
            return tot + all_v[pl.ds(i * _L, _L)]

        tot = lax.fori_loop(1, _NS, _fold, all_v[pl.ds(0, _L)])
        res = 1.0 - jnp.sum(tot) * (1.0 / _B)
        red_v[...] = jnp.full((_L,), res, jnp.float32)
        pltpu.sync_copy(red_v, out_hbm)


@jax.jit
def _loss(inT, tgt):
    mesh = plsc.VectorSubcoreMesh(
        core_axis_name="c", subcore_axis_name="s", num_cores=1
    )
    return pl.kernel(
        _loss_body,
        out_type=jax.ShapeDtypeStruct((_L,), jnp.float32),
        mesh=mesh,
        scratch_types=[
            pltpu.VMEM((_PER_TILE,), jnp.int32),
            pltpu.VMEM((_PER_TILE, 128), jnp.float32),
            pltpu.VMEM((_NS * _L,), jnp.float32),
            pltpu.VMEM((_L,), jnp.float32),
            pltpu.VMEM_SHARED((_NS * _L,), jnp.float32),
            pltpu.SemaphoreType.DMA,
        ],
        compiler_params=pltpu.CompilerParams(needs_layout_passes=False),
    )(inT, tgt)


def kernel(input_, target):
    out = _loss(input_.T, target.astype(jnp.int32))
    return out[0]


# speedup vs baseline: 1.3188x; 1.0056x over previous
import functools

import jax
import jax.numpy as jnp
from jax import lax
from jax.experimental import pallas as pl
from jax.experimental.pallas import tpu as pltpu
from jax.experimental.pallas import tpu_sc as plsc

_B = 1024
_V = 100000
_L = 16
_NS = 16
_PER_TILE = _B // _NS
_CHUNKS = _PER_TILE // _L


def _loss_body(inT_hbm, tgt_hbm, out_hbm, tgt_v, val_v, all_v, red_v, shared, sem):
    sid = lax.axis_index("s")
    base = sid * _PER_TILE

    pltpu.sync_copy(tgt_hbm.at[pl.ds(base, _PER_TILE)], tgt_v)

    col0 = pl.multiple_of((sid // 2) * 128, 128)
    pltpu.async_copy(
        inT_hbm.at[tgt_v, pl.ds(col0, 128)], val_v, sem
    ).wait()

    acc = jnp.zeros((_L,), jnp.float32)
    cbase = (sid % 2) * _PER_TILE
    for j in range(_CHUNKS):
        row = lax.iota(jnp.int32, _L) + (j * _L)
        col = lax.iota(jnp.int32, _L) + (cbase + j * _L)
        acc = acc + plsc.load_gather(val_v, [row, col])
    red_v[...] = acc
    pltpu.sync_copy(red_v, shared.at[pl.ds(sid * _L, _L)])
    plsc.subcore_barrier()

    @pl.when(sid == 0)
    def _():
        pltpu.sync_copy(shared, all_v)
        tot = all_v[pl.ds(0, _L)]
        for i in range(1, _NS):
            tot = tot + all_v[pl.ds(i * _L, _L)]
        res = 1.0 - jnp.sum(tot) * (1.0 / _B)
        red_v[...] = jnp.full((_L,), res, jnp.float32)
        pltpu.sync_copy(red_v, out_hbm)


@jax.jit
def _loss(inT, tgt):
    mesh = plsc.VectorSubcoreMesh(
        core_axis_name="c", subcore_axis_name="s", num_cores=1
    )
    return pl.kernel(
        _loss_body,
        out_type=jax.ShapeDtypeStruct((_L,), jnp.float32),
        mesh=mesh,
        scratch_types=[
            pltpu.VMEM((_PER_TILE,), jnp.int32),
            pltpu.VMEM((_PER_TILE, 128), jnp.float32),
            pltpu.VMEM((_NS * _L,), jnp.float32),
            pltpu.VMEM((_L,), jnp.float32),
            pltpu.VMEM_SHARED((_NS * _L,), jnp.float32),
            pltpu.SemaphoreType.DMA,
        ],
        compiler_params=pltpu.CompilerParams(needs_layout_passes=False),
    )(inT, tgt)


def kernel(input_, target):
    out = _loss(input_.T, target.astype(jnp.int32))
    return out[0]
